# baseline (speedup 1.0000x reference)
import functools

import jax
import jax.numpy as jnp
from jax import lax
from jax.experimental import pallas as pl
from jax.experimental.pallas import tpu as pltpu
from jax.experimental.pallas import tpu_sc as plsc

KMAX = 10
KMIN = 10

NUM_CORES = 2
NUM_SUBCORES = 16
LANES = 16
TILE = 16
GROUP = 8

_SORT8 = [(0, 1), (2, 3), (4, 5), (6, 7),
          (0, 2), (1, 3), (4, 6), (5, 7),
          (1, 2), (5, 6),
          (0, 4), (1, 5), (2, 6), (3, 7),
          (2, 4), (3, 5),
          (1, 2), (3, 4), (5, 6)]


def _sort8_desc(v):
    v = list(v)
    for i, j in _SORT8:
        hi = jnp.maximum(v[i], v[j])
        lo = jnp.minimum(v[i], v[j])
        v[i], v[j] = hi, lo
    return v


def _merge_top(T, A):
    C = list(T)
    for i in range(GROUP):
        C[8 + i] = jnp.maximum(T[8 + i], A[7 - i])
    for d in (8, 4, 2, 1):
        for j in range(16):
            if (j % (2 * d)) < d:
                hi = jnp.maximum(C[j], C[j + d])
                lo = jnp.minimum(C[j], C[j + d])
                C[j], C[j + d] = hi, lo
    return C


def _merge_bot(B, A):
    C = list(B)
    for i in range(GROUP):
        C[8 + i] = jnp.minimum(B[8 + i], A[i])
    for d in (8, 4, 2, 1):
        for j in range(16):
            if (j % (2 * d)) < d:
                lo = jnp.minimum(C[j], C[j + d])
                hi = jnp.maximum(C[j], C[j + d])
                C[j], C[j + d] = lo, hi
    return C


def _make_sc_kernel(rows, n):
    num_workers = NUM_CORES * NUM_SUBCORES
    rows_per_w = rows // num_workers
    tiles = rows_per_w // TILE
    groups = n // GROUP

    mesh = plsc.VectorSubcoreMesh(
        core_axis_name="c", subcore_axis_name="s",
        num_cores=NUM_CORES, num_subcores=NUM_SUBCORES)

    @functools.partial(
        pl.kernel,
        mesh=mesh,
        out_type=jax.ShapeDtypeStruct((rows,), jnp.float32),
        scratch_types=[
            pltpu.VMEM((TILE * n,), jnp.float32),
            pltpu.VMEM((rows_per_w,), jnp.float32),
        ],
        compiler_params=pltpu.CompilerParams(
            use_tc_tiling_on_sc=False, needs_layout_passes=False),
    )
    def k(x_hbm, out_hbm, buf_v, out_v):
        wid = lax.axis_index("s") * NUM_CORES + lax.axis_index("c")
        row0 = wid * rows_per_w
        lanes = lax.iota(jnp.int32, LANES)
        lanebase = lanes * n
        neg = jnp.full((LANES,), -jnp.inf, jnp.float32)
        pos = jnp.full((LANES,), jnp.inf, jnp.float32)

        def tile_body(t, _):
            pltpu.sync_copy(x_hbm.at[pl.ds((row0 + t * TILE) * n, TILE * n)],
                            buf_v)

            def group_body(g, carry):
                T = list(carry[:16])
                Bo = list(carry[16:])
                cb = lanebase + g * GROUP
                A = [plsc.load_gather(buf_v, [cb + kk])
                     for kk in range(GROUP)]
                A = _sort8_desc(A)
                T = _merge_top(T, A)
                Bo = _merge_bot(Bo, A)
                return tuple(T) + tuple(Bo)

            init = (neg,) * 16 + (pos,) * 16
            fin = lax.fori_loop(0, groups, group_body, init)
            top_sum = fin[0]
            for j in range(1, KMAX):
                top_sum = top_sum + fin[j]
            bot_sum = fin[16]
            for j in range(1, KMIN):
                bot_sum = bot_sum + fin[16 + j]
            res = (top_sum / KMAX + bot_sum / KMIN) * jnp.float32(0.5)
            out_v[pl.ds(t * TILE, TILE)] = res
            return 0

        lax.fori_loop(0, tiles, tile_body, 0)
        pltpu.sync_copy(out_v, out_hbm.at[pl.ds(row0, rows_per_w)])

    return k


def kernel(input):
    B, C, H, W = input.shape
    n = H * W
    rows = B * C
    x = input.reshape(rows * n)
    out = _make_sc_kernel(rows, n)(x)
    return out.reshape(B, C)


# speedup vs baseline: 1.0674x; 1.0674x over previous
import functools

import jax
import jax.numpy as jnp
from jax import lax
from jax.experimental import pallas as pl
from jax.experimental.pallas import tpu as pltpu
from jax.experimental.pallas import tpu_sc as plsc

KMAX = 10
KMIN = 10

NUM_CORES = 2
NUM_SUBCORES = 16
LANES = 16
TILE = 16
GROUP = 16


def _batcher(num):
    def oe_merge(lo, nn, r):
        step = r * 2
        if step < nn:
            yield from oe_merge(lo, nn, step)
            yield from oe_merge(lo + r, nn, step)
            for i in range(lo + r, lo + nn - r, step):
                yield (i, i + r)
        else:
            yield (lo, lo + r)

    def srt(lo, nn):
        if nn > 1:
            m = nn // 2
            yield from srt(lo, m)
            yield from srt(lo + m, m)
            yield from oe_merge(lo, nn, 1)

    return list(srt(0, num))


_SORT16 = _batcher(GROUP)


def _sort16_desc(v):
    v = list(v)
    for i, j in _SORT16:
        hi = jnp.maximum(v[i], v[j])
        lo = jnp.minimum(v[i], v[j])
        v[i], v[j] = hi, lo
    return v


def _merge_top(T, A):
    C = [jnp.maximum(T[i], A[15 - i]) for i in range(16)]
    for d in (8, 4, 2, 1):
        for j in range(16):
            if (j % (2 * d)) < d:
                hi = jnp.maximum(C[j], C[j + d])
                lo = jnp.minimum(C[j], C[j + d])
                C[j], C[j + d] = hi, lo
    return C


def _merge_bot(B, A):
    C = [jnp.minimum(B[i], A[i]) for i in range(16)]
    for d in (8, 4, 2, 1):
        for j in range(16):
            if (j % (2 * d)) < d:
                lo = jnp.minimum(C[j], C[j + d])
                hi = jnp.maximum(C[j], C[j + d])
                C[j], C[j + d] = lo, hi
    return C


def _make_sc_kernel(rows, n):
    num_workers = NUM_CORES * NUM_SUBCORES
    rows_per_w = rows // num_workers
    tiles = rows_per_w // TILE
    groups = n // GROUP

    mesh = plsc.VectorSubcoreMesh(
        core_axis_name="c", subcore_axis_name="s",
        num_cores=NUM_CORES, num_subcores=NUM_SUBCORES)

    @functools.partial(
        pl.kernel,
        mesh=mesh,
        out_type=jax.ShapeDtypeStruct((rows,), jnp.float32),
        scratch_types=[
            pltpu.VMEM((TILE * n,), jnp.float32),
            pltpu.VMEM((rows_per_w,), jnp.float32),
        ],
        compiler_params=pltpu.CompilerParams(
            use_tc_tiling_on_sc=False, needs_layout_passes=False),
    )
    def k(x_hbm, out_hbm, buf_v, out_v):
        wid = lax.axis_index("s") * NUM_CORES + lax.axis_index("c")
        row0 = wid * rows_per_w
        lanes = lax.iota(jnp.int32, LANES)
        lanebase = lanes * n
        neg = jnp.full((LANES,), -jnp.inf, jnp.float32)
        pos = jnp.full((LANES,), jnp.inf, jnp.float32)

        def tile_body(t, _):
            pltpu.sync_copy(x_hbm.at[pl.ds((row0 + t * TILE) * n, TILE * n)],
                            buf_v)

            def group_body(g, carry):
                T = list(carry[:16])
                Bo = list(carry[16:32])
                iv = carry[32]
                A = [plsc.load_gather(buf_v, [iv + kk])
                     for kk in range(GROUP)]
                A = _sort16_desc(A)
                T = _merge_top(T, A)
                Bo = _merge_bot(Bo, A)
                return tuple(T) + tuple(Bo) + (iv + GROUP,)

            init = (neg,) * 16 + (pos,) * 16 + (lanebase,)
            fin = lax.fori_loop(0, groups, group_body, init)
            top_sum = fin[0]
            for j in range(1, KMAX):
                top_sum = top_sum + fin[j]
            bot_sum = fin[16]
            for j in range(1, KMIN):
                bot_sum = bot_sum + fin[16 + j]
            res = (top_sum / KMAX + bot_sum / KMIN) * jnp.float32(0.5)
            out_v[pl.ds(t * TILE, TILE)] = res
            return 0

        lax.fori_loop(0, tiles, tile_body, 0)
        pltpu.sync_copy(out_v, out_hbm.at[pl.ds(row0, rows_per_w)])

    return k


def kernel(input):
    B, C, H, W = input.shape
    n = H * W
    rows = B * C
    x = input.reshape(rows * n)
    out = _make_sc_kernel(rows, n)(x)
    return out.reshape(B, C)


# speedup vs baseline: 1.5467x; 1.4490x over previous
import functools

import jax
import jax.numpy as jnp
from jax import lax
from jax.experimental import pallas as pl
from jax.experimental.pallas import tpu as pltpu
from jax.experimental.pallas import tpu_sc as plsc

KMAX = 10
KMIN = 10

NUM_CORES = 2
NUM_SUBCORES = 16
LANES = 16
TILE = 16
GROUP = 16


def _batcher(num):
    def oe_merge(lo, nn, r):
        step = r * 2
        if step < nn:
            yield from oe_merge(lo, nn, step)
            yield from oe_merge(lo + r, nn, step)
            for i in range(lo + r, lo + nn - r, step):
                yield (i, i + r)
        else:
            yield (lo, lo + r)

    def srt(lo, nn):
        if nn > 1:
            m = nn // 2
            yield from srt(lo, m)
            yield from srt(lo + m, m)
            yield from oe_merge(lo, nn, 1)

    return list(srt(0, num))


_SORT16 = _batcher(GROUP)


def _sort16_desc(v):
    v = list(v)
    for i, j in _SORT16:
        hi = jnp.maximum(v[i], v[j])
        lo = jnp.minimum(v[i], v[j])
        v[i], v[j] = hi, lo
    return v


def _merge_top(T, A):
    C = [jnp.maximum(T[i], A[15 - i]) for i in range(16)]
    for d in (8, 4, 2, 1):
        for j in range(16):
            if (j % (2 * d)) < d:
                hi = jnp.maximum(C[j], C[j + d])
                lo = jnp.minimum(C[j], C[j + d])
                C[j], C[j + d] = hi, lo
    return C


def _merge_bot(B, A):
    C = [jnp.minimum(B[i], A[i]) for i in range(16)]
    for d in (8, 4, 2, 1):
        for j in range(16):
            if (j % (2 * d)) < d:
                lo = jnp.minimum(C[j], C[j + d])
                hi = jnp.maximum(C[j], C[j + d])
                C[j], C[j + d] = lo, hi
    return C


def _make_sc_kernel(rows, n):
    num_workers = NUM_CORES * NUM_SUBCORES
    rows_per_w = rows // num_workers
    tiles = rows_per_w // TILE
    groups = n // GROUP

    mesh = plsc.VectorSubcoreMesh(
        core_axis_name="c", subcore_axis_name="s",
        num_cores=NUM_CORES, num_subcores=NUM_SUBCORES)

    @functools.partial(
        pl.kernel,
        mesh=mesh,
        out_type=jax.ShapeDtypeStruct((rows,), jnp.float32),
        scratch_types=[
            pltpu.VMEM((TILE * n,), jnp.float32),
            pltpu.VMEM((rows_per_w,), jnp.float32),
        ],
        compiler_params=pltpu.CompilerParams(
            use_tc_tiling_on_sc=False, needs_layout_passes=False),
    )
    def k(x_hbm, out_hbm, buf_v, out_v):
        wid = lax.axis_index("s") * NUM_CORES + lax.axis_index("c")
        row0 = wid * rows_per_w
        lanes = lax.iota(jnp.int32, LANES)
        lanebase = lanes * n
        rot = lanes * 17
        neg = jnp.full((LANES,), -jnp.inf, jnp.float32)
        pos = jnp.full((LANES,), jnp.inf, jnp.float32)

        def tile_body(t, _):
            pltpu.sync_copy(x_hbm.at[pl.ds((row0 + t * TILE) * n, TILE * n)],
                            buf_v)

            def group_body(g, carry):
                T = list(carry[:16])
                Bo = list(carry[16:32])
                iv = carry[32]
                A = [plsc.load_gather(
                        buf_v, [lanebase + ((iv + kk) & (n - 1))])
                     for kk in range(GROUP)]
                A = _sort16_desc(A)
                T = _merge_top(T, A)
                Bo = _merge_bot(Bo, A)
                return tuple(T) + tuple(Bo) + (iv + GROUP,)

            init = (neg,) * 16 + (pos,) * 16 + (rot,)
            fin = lax.fori_loop(0, groups, group_body, init)
            top_sum = fin[0]
            for j in range(1, KMAX):
                top_sum = top_sum + fin[j]
            bot_sum = fin[16]
            for j in range(1, KMIN):
                bot_sum = bot_sum + fin[16 + j]
            res = (top_sum / KMAX + bot_sum / KMIN) * jnp.float32(0.5)
            out_v[pl.ds(t * TILE, TILE)] = res
            return 0

        lax.fori_loop(0, tiles, tile_body, 0)
        pltpu.sync_copy(out_v, out_hbm.at[pl.ds(row0, rows_per_w)])

    return k


def kernel(input):
    B, C, H, W = input.shape
    n = H * W
    rows = B * C
    x = input.reshape(rows * n)
    out = _make_sc_kernel(rows, n)(x)
    return out.reshape(B, C)
